# baseline (speedup 1.0000x reference)
import functools

import jax
import jax.numpy as jnp
from jax import lax
from jax.experimental import pallas as pl
from jax.experimental.pallas import tpu as pltpu
from jax.experimental.pallas import tpu_sc as plsc

D = 128
DM = 2 * D
B = 16384

NC = 2
NS = 16
NW = NC * NS
CHW = 128
CHN = 128
SLICES = (4096, 4096, 4096, 4096)


_GATHER_CACHE = {}


def _sc_gather(sbase, nrows, inputs, eu_gmf, ei_gmf, eu_mlp, ei_mlp):
  sl = lax.slice(inputs, (0, sbase), (2, sbase + nrows))
  fn = _build_sc_gather(nrows)
  return fn(sl, eu_gmf, ei_gmf, eu_mlp, ei_mlp)


def _build_sc_gather(nrows):
  if nrows in _GATHER_CACHE:
    return _GATHER_CACHE[nrows]
  mesh = plsc.VectorSubcoreMesh(core_axis_name="c", subcore_axis_name="s")
  bpw = nrows // NW

  @functools.partial(
      pl.kernel,
      out_type=(
          jax.ShapeDtypeStruct((nrows, DM), jnp.float32),
          jax.ShapeDtypeStruct((nrows, DM), jnp.float32),
          jax.ShapeDtypeStruct((nrows, D), jnp.float32),
          jax.ShapeDtypeStruct((nrows, D), jnp.float32),
      ),
      mesh=mesh,
      scratch_types=[
          pltpu.VMEM((bpw,), jnp.int32),
          pltpu.VMEM((bpw,), jnp.int32),
          pltpu.VMEM((CHW, DM), jnp.float32),
          pltpu.VMEM((CHW, DM), jnp.float32),
          pltpu.VMEM((CHN, D), jnp.float32),
          pltpu.VMEM((CHN, D), jnp.float32),
          pltpu.SemaphoreType.DMA,
          pltpu.SemaphoreType.DMA,
          pltpu.SemaphoreType.DMA,
          pltpu.SemaphoreType.DMA,
      ],
  )
  def body(idx_hbm, eu_gmf_h, ei_gmf_h, eu_mlp_h, ei_mlp_h,
           um_out, im_out, ug_out, ig_out,
           idx_u, idx_s, bw0, bw1, bn0, bn1, sw0, sw1, sn0, sn1):
    wid = lax.axis_index("s") * NC + lax.axis_index("c")
    base = wid * bpw
    du = pltpu.async_copy(idx_hbm.at[0, pl.ds(base, bpw)], idx_u, sn0)
    ds = pltpu.async_copy(idx_hbm.at[1, pl.ds(base, bpw)], idx_s, sn1)
    du.wait()
    ds.wait()

    tasks = []
    for tbl, idx, out, ch, bufs, sems in (
        (eu_mlp_h, idx_u, um_out, CHW, (bw0, bw1), (sw0, sw1)),
        (ei_mlp_h, idx_s, im_out, CHW, (bw0, bw1), (sw0, sw1)),
        (eu_gmf_h, idx_u, ug_out, CHN, (bn0, bn1), (sn0, sn1)),
        (ei_gmf_h, idx_s, ig_out, CHN, (bn0, bn1), (sn0, sn1)),
    ):
      for ci in range(bpw // ch):
        tasks.append((tbl, idx, out, ch, ci, bufs, sems))
    slot_of = {}
    counters = {}
    for t, (_, _, _, ch, _, _, _) in enumerate(tasks):
      slot_of[t] = counters.get(ch, 0) % 2
      counters[ch] = counters.get(ch, 0) + 1
    descs = [None] * len(tasks)

    def start(t):
      tbl, idx, out, ch, ci, bufs, sems = tasks[t]
      s = slot_of[t]
      descs[t] = pltpu.async_copy(
          tbl.at[idx.at[pl.ds(ci * ch, ch)]], bufs[s], sems[s])

    start(0)
    for t in range(len(tasks)):
      if t + 1 < len(tasks):
        start(t + 1)
      descs[t].wait()
      _, _, out, ch, ci, bufs, _ = tasks[t]
      pltpu.sync_copy(bufs[slot_of[t]], out.at[pl.ds(base + ci * ch, ch)])

  _GATHER_CACHE[nrows] = body
  return body


def _dense_body(um_ref, im_ref, ug_ref, ig_ref,
                w1u_ref, w1i_ref, b1_ref, w2_ref, b2_ref, wp_ref, bp_ref,
                *rest):
  emb_ref, y_ref = rest[-2], rest[-1]
  bf = jnp.bfloat16
  h = jnp.dot(um_ref[...].astype(bf), w1u_ref[...].astype(bf),
              preferred_element_type=jnp.float32)
  h += jnp.dot(im_ref[...].astype(bf), w1i_ref[...].astype(bf),
               preferred_element_type=jnp.float32)
  h = jnp.maximum(h + b1_ref[...], 0.0)
  h2 = jnp.dot(h.astype(bf), w2_ref[...].astype(bf),
               preferred_element_type=jnp.float32)
  h2 = jnp.maximum(h2 + b2_ref[...], 0.0)
  gmf = ug_ref[...] * ig_ref[...]
  emb = jnp.concatenate([gmf, h2], axis=-1)
  emb_ref[...] = emb
  y_ref[...] = jnp.sum(emb * wp_ref[...], axis=-1) + bp_ref[0]


def _tc_dense_slice(row_off, nrows, um, im, ug, ig,
                    w1u, w1i, b1r, w2, b2r, wpr, bpr, emb_in, y_in):
  bs = 2048
  nblk = nrows // bs
  in_specs = [
      pl.BlockSpec((bs, DM), lambda i: (i, 0)),
      pl.BlockSpec((bs, DM), lambda i: (i, 0)),
      pl.BlockSpec((bs, D), lambda i: (i, 0)),
      pl.BlockSpec((bs, D), lambda i: (i, 0)),
      pl.BlockSpec((DM, DM), lambda i: (0, 0)),
      pl.BlockSpec((DM, DM), lambda i: (0, 0)),
      pl.BlockSpec((1, DM), lambda i: (0, 0)),
      pl.BlockSpec((DM, D), lambda i: (0, 0)),
      pl.BlockSpec((1, D), lambda i: (0, 0)),
      pl.BlockSpec((1, DM), lambda i: (0, 0)),
      pl.BlockSpec(memory_space=pltpu.SMEM),
  ]
  args = [um, im, ug, ig, w1u, w1i, b1r, w2, b2r, wpr, bpr]
  aliases = {}
  if row_off > 0:
    in_specs += [pl.BlockSpec(memory_space=pl.ANY),
                 pl.BlockSpec(memory_space=pl.ANY)]
    args += [emb_in, y_in]
    aliases = {11: 0, 12: 1}
  off = row_off // bs
  return pl.pallas_call(
      _dense_body,
      grid=(nblk,),
      in_specs=in_specs,
      out_specs=[
          pl.BlockSpec((bs, DM), lambda i: (i + off, 0)),
          pl.BlockSpec((bs,), lambda i: (i + off,)),
      ],
      out_shape=[
          jax.ShapeDtypeStruct((B, DM), jnp.float32),
          jax.ShapeDtypeStruct((B,), jnp.float32),
      ],
      input_output_aliases=aliases,
  )(*args)


def kernel(inputs, eu_gmf, ei_gmf, eu_mlp, ei_mlp, W1, b1, W2, b2, Wp, bp):
  w1t = W1.T
  w1u = w1t[:DM]
  w1i = w1t[DM:]
  w2 = W2.T
  b1r = b1.reshape(1, -1)
  b2r = b2.reshape(1, -1)
  wpr = Wp.reshape(1, -1)
  bpr = bp.reshape(1)
  emb, y = None, None
  off = 0
  for nrows in SLICES:
    um, im, ug, ig = _sc_gather(off, nrows, inputs,
                                eu_gmf, ei_gmf, eu_mlp, ei_mlp)
    emb, y = _tc_dense_slice(off, nrows, um, im, ug, ig,
                             w1u, w1i, b1r, w2, b2r, wpr, bpr, emb, y)
    off += nrows
  return emb, y


# speedup vs baseline: 1.0428x; 1.0428x over previous
import functools

import jax
import jax.numpy as jnp
from jax import lax
from jax.experimental import pallas as pl
from jax.experimental.pallas import tpu as pltpu
from jax.experimental.pallas import tpu_sc as plsc

D = 128
DM = 2 * D
B = 16384

NC = 2
NS = 16
NW = NC * NS
CHW = 128
CHN = 128
SLICES = (8192, 8192)


def _sc_gather(sbase, nrows, inputs, eu_gmf, ei_gmf, eu_mlp, ei_mlp):
  mesh = plsc.VectorSubcoreMesh(core_axis_name="c", subcore_axis_name="s")
  bpw = nrows // NW

  @functools.partial(
      pl.kernel,
      out_type=(
          jax.ShapeDtypeStruct((nrows, DM), jnp.float32),
          jax.ShapeDtypeStruct((nrows, DM), jnp.float32),
          jax.ShapeDtypeStruct((nrows, D), jnp.float32),
          jax.ShapeDtypeStruct((nrows, D), jnp.float32),
      ),
      mesh=mesh,
      scratch_types=[
          pltpu.VMEM((bpw,), jnp.int32),
          pltpu.VMEM((bpw,), jnp.int32),
          pltpu.VMEM((CHW, DM), jnp.float32),
          pltpu.VMEM((CHW, DM), jnp.float32),
          pltpu.VMEM((CHN, D), jnp.float32),
          pltpu.VMEM((CHN, D), jnp.float32),
          pltpu.SemaphoreType.DMA,
          pltpu.SemaphoreType.DMA,
          pltpu.SemaphoreType.DMA,
          pltpu.SemaphoreType.DMA,
      ],
  )
  def body(idx_hbm, eu_gmf_h, ei_gmf_h, eu_mlp_h, ei_mlp_h,
           um_out, im_out, ug_out, ig_out,
           idx_u, idx_s, bw0, bw1, bn0, bn1, sw0, sw1, sn0, sn1):
    wid = lax.axis_index("s") * NC + lax.axis_index("c")
    base = wid * bpw
    du = pltpu.async_copy(idx_hbm.at[0, pl.ds(sbase + base, bpw)], idx_u, sn0)
    ds = pltpu.async_copy(idx_hbm.at[1, pl.ds(sbase + base, bpw)], idx_s, sn1)
    du.wait()
    ds.wait()

    tasks = []
    for tbl, idx, out, ch, bufs, sems in (
        (eu_mlp_h, idx_u, um_out, CHW, (bw0, bw1), (sw0, sw1)),
        (ei_mlp_h, idx_s, im_out, CHW, (bw0, bw1), (sw0, sw1)),
        (eu_gmf_h, idx_u, ug_out, CHN, (bn0, bn1), (sn0, sn1)),
        (ei_gmf_h, idx_s, ig_out, CHN, (bn0, bn1), (sn0, sn1)),
    ):
      for ci in range(bpw // ch):
        tasks.append((tbl, idx, out, ch, ci, bufs, sems))
    slot_of = {}
    counters = {}
    for t, (_, _, _, ch, _, _, _) in enumerate(tasks):
      slot_of[t] = counters.get(ch, 0) % 2
      counters[ch] = counters.get(ch, 0) + 1
    descs = [None] * len(tasks)

    def start(t):
      tbl, idx, out, ch, ci, bufs, sems = tasks[t]
      s = slot_of[t]
      descs[t] = pltpu.async_copy(
          tbl.at[idx.at[pl.ds(ci * ch, ch)]], bufs[s], sems[s])

    start(0)
    for t in range(len(tasks)):
      if t + 1 < len(tasks):
        start(t + 1)
      descs[t].wait()
      _, _, out, ch, ci, bufs, _ = tasks[t]
      pltpu.sync_copy(bufs[slot_of[t]], out.at[pl.ds(base + ci * ch, ch)])

  return body(inputs, eu_gmf, ei_gmf, eu_mlp, ei_mlp)


def _dense_body(um_ref, im_ref, ug_ref, ig_ref,
                w1u_ref, w1i_ref, b1_ref, w2_ref, b2_ref, wp_ref, bp_ref,
                *rest):
  emb_ref, y_ref = rest[-2], rest[-1]
  bf = jnp.bfloat16
  h = jnp.dot(um_ref[...].astype(bf), w1u_ref[...].astype(bf),
              preferred_element_type=jnp.float32)
  h += jnp.dot(im_ref[...].astype(bf), w1i_ref[...].astype(bf),
               preferred_element_type=jnp.float32)
  h = jnp.maximum(h + b1_ref[...], 0.0)
  h2 = jnp.dot(h.astype(bf), w2_ref[...].astype(bf),
               preferred_element_type=jnp.float32)
  h2 = jnp.maximum(h2 + b2_ref[...], 0.0)
  gmf = ug_ref[...] * ig_ref[...]
  emb = jnp.concatenate([gmf, h2], axis=-1)
  emb_ref[...] = emb
  y_ref[...] = jnp.sum(emb * wp_ref[...], axis=-1) + bp_ref[0]


def _tc_dense_slice(row_off, nrows, um, im, ug, ig,
                    w1u, w1i, b1r, w2, b2r, wpr, bpr, emb_in, y_in):
  bs = 2048
  nblk = nrows // bs
  in_specs = [
      pl.BlockSpec((bs, DM), lambda i: (i, 0)),
      pl.BlockSpec((bs, DM), lambda i: (i, 0)),
      pl.BlockSpec((bs, D), lambda i: (i, 0)),
      pl.BlockSpec((bs, D), lambda i: (i, 0)),
      pl.BlockSpec((DM, DM), lambda i: (0, 0)),
      pl.BlockSpec((DM, DM), lambda i: (0, 0)),
      pl.BlockSpec((1, DM), lambda i: (0, 0)),
      pl.BlockSpec((DM, D), lambda i: (0, 0)),
      pl.BlockSpec((1, D), lambda i: (0, 0)),
      pl.BlockSpec((1, DM), lambda i: (0, 0)),
      pl.BlockSpec(memory_space=pltpu.SMEM),
  ]
  args = [um, im, ug, ig, w1u, w1i, b1r, w2, b2r, wpr, bpr]
  aliases = {}
  if row_off > 0:
    in_specs += [pl.BlockSpec(memory_space=pl.ANY),
                 pl.BlockSpec(memory_space=pl.ANY)]
    args += [emb_in, y_in]
    aliases = {11: 0, 12: 1}
  off = row_off // bs
  return pl.pallas_call(
      _dense_body,
      grid=(nblk,),
      in_specs=in_specs,
      out_specs=[
          pl.BlockSpec((bs, DM), lambda i: (i + off, 0)),
          pl.BlockSpec((bs,), lambda i: (i + off,)),
      ],
      out_shape=[
          jax.ShapeDtypeStruct((B, DM), jnp.float32),
          jax.ShapeDtypeStruct((B,), jnp.float32),
      ],
      input_output_aliases=aliases,
  )(*args)


def kernel(inputs, eu_gmf, ei_gmf, eu_mlp, ei_mlp, W1, b1, W2, b2, Wp, bp):
  w1t = W1.T
  w1u = w1t[:DM]
  w1i = w1t[DM:]
  w2 = W2.T
  b1r = b1.reshape(1, -1)
  b2r = b2.reshape(1, -1)
  wpr = Wp.reshape(1, -1)
  bpr = bp.reshape(1)
  emb, y = None, None
  off = 0
  for nrows in SLICES:
    um, im, ug, ig = _sc_gather(off, nrows, inputs,
                                eu_gmf, ei_gmf, eu_mlp, ei_mlp)
    emb, y = _tc_dense_slice(off, nrows, um, im, ug, ig,
                             w1u, w1i, b1r, w2, b2r, wpr, bpr, emb, y)
    off += nrows
  return emb, y
